# baseline (speedup 1.0000x reference)
import functools

import jax
import jax.numpy as jnp
import numpy as np
from jax import lax
from jax.experimental import pallas as pl
from jax.experimental.pallas import tpu as pltpu
from jax.experimental.pallas import tpu_sc as plsc

_VOCAB = 100000
_D = 128
_MAX_LEN = 2048
_B = 16
_L = 2048
_SCALE = float(np.sqrt(float(_D)))

_NW = 32
_POS = _L // _NW
_NBC = 4
_NCHUNK = _B // _NBC
_CROWS = _NBC * _POS
_LANES = 16


def _positional_rows():
    half = _D // 2
    positions = np.arange(_MAX_LEN)[:, None].astype(np.float32)
    depths = np.arange(half, dtype=np.float32)[None, :] / float(half)
    angle_rates = 1.0 / np.power(10000.0, depths)
    angle_rads = positions * angle_rates
    pe = np.concatenate([np.sin(angle_rads), np.cos(angle_rads)], axis=-1)
    return jnp.asarray(pe.reshape(-1), dtype=jnp.float32)


_MESH = plsc.VectorSubcoreMesh(
    core_axis_name="c", subcore_axis_name="s", num_cores=2, num_subcores=16
)


@functools.partial(
    pl.kernel,
    out_type=jax.ShapeDtypeStruct((_B * _L, _D), jnp.float32),
    mesh=_MESH,
    scratch_types=[
        pltpu.VMEM((_NCHUNK, _CROWS), jnp.int32),
        pltpu.VMEM((_CROWS, _D), jnp.float32),
        pltpu.VMEM((_CROWS, _D), jnp.float32),
        pltpu.VMEM((_POS * _D,), jnp.float32),
        pltpu.SemaphoreType.DMA,
        pltpu.SemaphoreType.DMA,
        pltpu.SemaphoreType.DMA,
        pltpu.SemaphoreType.DMA,
    ],
    compiler_params=pltpu.CompilerParams(use_tc_tiling_on_sc=False),
)
def _emb_kernel(x_hbm, table_hbm, pe_hbm, out_hbm,
                idx_v, buf0, buf1, pe_v, sem_g0, sem_g1, sem_s, sem_i):
    wid = lax.axis_index("s") * 2 + lax.axis_index("c")
    idx_fetches = [
        pltpu.make_async_copy(
            x_hbm.at[b, wid // 2, pl.ds((wid % 2) * _POS, _POS)],
            idx_v.at[b // _NBC, pl.ds((b % _NBC) * _POS, _POS)],
            sem_i,
        )
        for b in range(_B)
    ]
    for d in idx_fetches:
        d.start()
    pltpu.sync_copy(pe_hbm.at[pl.ds(wid * _POS * _D, _POS * _D)], pe_v)
    for d in idx_fetches:
        d.wait()

    bufs = [buf0, buf1]
    sems = [sem_g0, sem_g1]
    gathers = [
        pltpu.make_async_copy(table_hbm.at[idx_v.at[c]], bufs[c & 1], sems[c & 1])
        for c in range(_NCHUNK)
    ]
    stores = [[] for _ in range(_NCHUNK)]

    gathers[0].start()
    for c in range(_NCHUNK):
        if c + 1 < _NCHUNK:
            for s in stores[c - 1] if c >= 1 else ():
                s.wait()
            gathers[c + 1].start()
        gathers[c].wait()
        buf = bufs[c & 1]

        def row_body(r, carry, buf=buf):
            for j in range(_D // _LANES):
                sl = pl.ds(j * _LANES, _LANES)
                p = pe_v[pl.ds(r * _D + j * _LANES, _LANES)]
                for bb in range(_NBC):
                    buf[bb * _POS + r, sl] = buf[bb * _POS + r, sl] * _SCALE + p
            return carry

        lax.fori_loop(0, _POS, row_body, 0, unroll=False)

        for bb in range(_NBC):
            off = (c * _NBC + bb) * _L + wid * _POS
            d = pltpu.make_async_copy(
                buf.at[pl.ds(bb * _POS, _POS)], out_hbm.at[pl.ds(off, _POS)], sem_s
            )
            d.start()
            stores[c].append(d)

    for c in (_NCHUNK - 2, _NCHUNK - 1):
        for s in stores[c]:
            s.wait()


_PE_ROWS = _positional_rows()


def kernel(x, table):
    pe = _PE_ROWS + table[0, 0] * 0.0
    x3 = x.astype(jnp.int32).reshape(_B, _L // _D, _D)
    out = _emb_kernel(x3, table, pe)
    return out.reshape(_B, _L, _D)


# speedup vs baseline: 1.0448x; 1.0448x over previous
import functools

import jax
import jax.numpy as jnp
import numpy as np
from jax import lax
from jax.experimental import pallas as pl
from jax.experimental.pallas import tpu as pltpu
from jax.experimental.pallas import tpu_sc as plsc

_VOCAB = 100000
_D = 128
_MAX_LEN = 2048
_B = 16
_L = 2048
_SCALE = float(np.sqrt(float(_D)))

_NW = 32
_POS = _L // _NW
_NBC = 4
_NCHUNK = _B // _NBC
_CROWS = _NBC * _POS
_LANES = 16


def _positional_rows():
    half = _D // 2
    positions = np.arange(_MAX_LEN)[:, None].astype(np.float32)
    depths = np.arange(half, dtype=np.float32)[None, :] / float(half)
    angle_rates = 1.0 / np.power(10000.0, depths)
    angle_rads = positions * angle_rates
    pe = np.concatenate([np.sin(angle_rads), np.cos(angle_rads)], axis=-1)
    return jnp.asarray(pe.reshape(-1), dtype=jnp.float32)


_MESH = plsc.VectorSubcoreMesh(
    core_axis_name="c", subcore_axis_name="s", num_cores=2, num_subcores=16
)


@functools.partial(
    pl.kernel,
    out_type=jax.ShapeDtypeStruct((_B * _L, _D), jnp.float32),
    mesh=_MESH,
    scratch_types=[
        pltpu.VMEM((_NCHUNK, _CROWS), jnp.int32),
        pltpu.VMEM((_CROWS, _D), jnp.float32),
        pltpu.VMEM((_CROWS, _D), jnp.float32),
        pltpu.VMEM((_CROWS, _D), jnp.float32),
        pltpu.VMEM((_POS * _D,), jnp.float32),
        pltpu.SemaphoreType.DMA,
        pltpu.SemaphoreType.DMA,
        pltpu.SemaphoreType.DMA,
        pltpu.SemaphoreType.DMA,
        pltpu.SemaphoreType.DMA,
    ],
    compiler_params=pltpu.CompilerParams(use_tc_tiling_on_sc=False),
)
def _emb_kernel(x_hbm, table_hbm, pe_hbm, out_hbm,
                idx_v, buf0, buf1, buf2, pe_v,
                sem_g0, sem_g1, sem_g2, sem_s, sem_i):
    wid = lax.axis_index("s") * 2 + lax.axis_index("c")
    idx_fetches = [
        pltpu.make_async_copy(
            x_hbm.at[b, wid // 2, pl.ds((wid % 2) * _POS, _POS)],
            idx_v.at[b // _NBC, pl.ds((b % _NBC) * _POS, _POS)],
            sem_i,
        )
        for b in range(_B)
    ]
    for d in idx_fetches:
        d.start()
    for d in idx_fetches:
        d.wait()

    bufs = [buf0, buf1, buf2]
    sems = [sem_g0, sem_g1, sem_g2]
    gathers = [
        pltpu.make_async_copy(table_hbm.at[idx_v.at[c]], bufs[c % 3], sems[c % 3])
        for c in range(_NCHUNK)
    ]
    stores = [[] for _ in range(_NCHUNK)]

    gathers[0].start()
    pltpu.sync_copy(pe_hbm.at[pl.ds(wid * _POS * _D, _POS * _D)], pe_v)
    for c in range(_NCHUNK):
        if c + 1 < _NCHUNK:
            for s in stores[c - 2] if c >= 2 else ():
                s.wait()
            gathers[c + 1].start()
        gathers[c].wait()
        buf = bufs[c % 3]

        def row_body(r, carry, buf=buf):
            for j in range(_D // _LANES):
                sl = pl.ds(j * _LANES, _LANES)
                p = pe_v[pl.ds(r * _D + j * _LANES, _LANES)]
                for bb in range(_NBC):
                    buf[bb * _POS + r, sl] = buf[bb * _POS + r, sl] * _SCALE + p
            return carry

        lax.fori_loop(0, _POS, row_body, 0, unroll=False)

        for bb in range(_NBC):
            off = (c * _NBC + bb) * _L + wid * _POS
            d = pltpu.make_async_copy(
                buf.at[pl.ds(bb * _POS, _POS)], out_hbm.at[pl.ds(off, _POS)], sem_s
            )
            d.start()
            stores[c].append(d)

    for c in (_NCHUNK - 3, _NCHUNK - 2, _NCHUNK - 1):
        for s in stores[c]:
            s.wait()


_PE_ROWS = _positional_rows()


def kernel(x, table):
    pe = _PE_ROWS + table[0, 0] * 0.0
    x3 = x.astype(jnp.int32).reshape(_B, _L // _D, _D)
    out = _emb_kernel(x3, table, pe)
    return out.reshape(_B, _L, _D)
